# baseline (speedup 1.0000x reference)
import jax
import jax.numpy as jnp
from jax.experimental import pallas as pl
from jax.experimental.pallas import tpu as pltpu
from jax.experimental.pallas import tpu_sc as plsc

VOCAB = 1000000
HIDDEN = 128
SEQ = 50
EPS = 1e-12

GATHER_WINDOW = 256
B_BLK = 256


def _sc_gather(word_emb, ids_flat, n_rows):
    vector_mesh = plsc.VectorSubcoreMesh(
        core_axis_name="core", subcore_axis_name="subcore"
    )

    @pl.kernel(
        out_type=jax.ShapeDtypeStruct((n_rows, HIDDEN), word_emb.dtype),
        mesh=vector_mesh,
    )
    def gather_kernel(x_hbm, i_hbm, o_hbm):
        def body(i_vmem, o_vmem):
            pltpu.sync_copy(x_hbm.at[i_vmem.at[0]], o_vmem)

        pltpu.emit_pipeline(
            body,
            grid=(n_rows // GATHER_WINDOW,),
            in_specs=[
                pl.BlockSpec((1, GATHER_WINDOW), index_map=lambda i: (0, i))
            ],
            out_specs=[
                pl.BlockSpec((GATHER_WINDOW, HIDDEN), index_map=lambda i: (i, 0))
            ],
            core_axis_name=("core", "subcore"),
            dimension_semantics=(pltpu.PARALLEL,),
        )(i_hbm, o_hbm)

    return gather_kernel(word_emb, ids_flat)


def _tc_add_ln_kernel(x_ref, comb_ref, gamma_ref, beta_ref, o_ref):
    x = x_ref[...]
    e = x + comb_ref[...]
    mean = jnp.mean(e, axis=-1, keepdims=True)
    cent = e - mean
    var = jnp.mean(cent * cent, axis=-1, keepdims=True)
    inv = jax.lax.rsqrt(var + EPS)
    o_ref[...] = cent * inv * gamma_ref[...] + beta_ref[...]


def _tc_add_ln_chunk(gathered_sm, comb, gamma, beta, batch, seq, seq_chunk,
                     chunk_idx, out_alias=None):
    grid = (batch // B_BLK,)
    args = [gathered_sm, comb, gamma, beta]
    in_specs = [
        pl.BlockSpec((seq_chunk, B_BLK, HIDDEN), lambda i: (0, i, 0)),
        pl.BlockSpec((seq_chunk, 1, HIDDEN), lambda i: (0, 0, 0)),
        pl.BlockSpec((1, 1, HIDDEN), lambda i: (0, 0, 0)),
        pl.BlockSpec((1, 1, HIDDEN), lambda i: (0, 0, 0)),
    ]
    kwargs = {}
    if out_alias is not None:
        args.append(out_alias)
        in_specs.append(
            pl.BlockSpec(memory_space=pltpu.MemorySpace.HBM)
        )
        kwargs["input_output_aliases"] = {4: 0}

    def body(*refs):
        _tc_add_ln_kernel(*refs[:4], refs[-1])

    return pl.pallas_call(
        body,
        grid=grid,
        in_specs=in_specs,
        out_specs=pl.BlockSpec(
            (seq_chunk, B_BLK, HIDDEN), lambda i, c=chunk_idx: (c, i, 0)
        ),
        out_shape=jax.ShapeDtypeStruct((seq, batch, HIDDEN), jnp.float32),
        compiler_params=pltpu.CompilerParams(
            dimension_semantics=("parallel",)
        ),
        **kwargs,
    )(*args)


def kernel(input_ids, word_emb, pos_emb, type_emb, gamma, beta):
    batch, seq = input_ids.shape
    n_chunks = 5
    seq_chunk = seq // n_chunks
    rows_chunk = batch * seq_chunk
    ids_sm = input_ids.astype(jnp.int32).T.reshape(1, batch * seq)
    comb = (pos_emb[:seq] + type_emb[0:1, :]).reshape(seq, 1, HIDDEN)
    gamma3 = gamma.reshape(1, 1, HIDDEN)
    beta3 = beta.reshape(1, 1, HIDDEN)

    gathered = [
        _sc_gather(
            word_emb,
            jax.lax.slice(ids_sm, (0, c * rows_chunk), (1, (c + 1) * rows_chunk)),
            rows_chunk,
        ).reshape(seq_chunk, batch, HIDDEN)
        for c in range(n_chunks)
    ]
    out = None
    for c in range(n_chunks):
        comb_c = jax.lax.slice(
            comb, (c * seq_chunk, 0, 0), ((c + 1) * seq_chunk, 1, HIDDEN)
        )
        out = _tc_add_ln_chunk(
            gathered[c], comb_c, gamma3, beta3, batch, seq, seq_chunk,
            chunk_idx=c, out_alias=out,
        )
    return jnp.transpose(out, (1, 0, 2))


# speedup vs baseline: 1.0741x; 1.0741x over previous
import jax
import jax.numpy as jnp
from jax.experimental import pallas as pl
from jax.experimental.pallas import tpu as pltpu
from jax.experimental.pallas import tpu_sc as plsc

VOCAB = 1000000
HIDDEN = 128
SEQ = 50
EPS = 1e-12

GATHER_WINDOW = 256
B_BLK = 256


def _sc_gather(word_emb, ids_flat, n_rows):
    vector_mesh = plsc.VectorSubcoreMesh(
        core_axis_name="core", subcore_axis_name="subcore"
    )

    @pl.kernel(
        out_type=jax.ShapeDtypeStruct((n_rows, HIDDEN), word_emb.dtype),
        mesh=vector_mesh,
    )
    def gather_kernel(x_hbm, i_hbm, o_hbm):
        def body(i_vmem, o_vmem):
            pltpu.sync_copy(x_hbm.at[i_vmem.at[0]], o_vmem)

        pltpu.emit_pipeline(
            body,
            grid=(n_rows // GATHER_WINDOW,),
            in_specs=[
                pl.BlockSpec((1, GATHER_WINDOW), index_map=lambda i: (0, i))
            ],
            out_specs=[
                pl.BlockSpec((GATHER_WINDOW, HIDDEN), index_map=lambda i: (i, 0))
            ],
            core_axis_name=("core", "subcore"),
            dimension_semantics=(pltpu.PARALLEL,),
        )(i_hbm, o_hbm)

    return gather_kernel(word_emb, ids_flat)


def _tc_add_ln_kernel(x_ref, comb_ref, gamma_ref, beta_ref, o_ref):
    x = x_ref[...]
    e = x + comb_ref[...]
    mean = jnp.mean(e, axis=-1, keepdims=True)
    cent = e - mean
    var = jnp.mean(cent * cent, axis=-1, keepdims=True)
    inv = jax.lax.rsqrt(var + EPS)
    o_ref[...] = cent * inv * gamma_ref[...] + beta_ref[...]


def _tc_add_ln_chunk(gathered_sm, comb, gamma, beta, batch, seq, seq_chunk,
                     chunk_idx, out_alias=None):
    grid = (batch // B_BLK,)
    args = [gathered_sm, comb, gamma, beta]
    in_specs = [
        pl.BlockSpec((seq_chunk, B_BLK, HIDDEN), lambda i: (0, i, 0)),
        pl.BlockSpec((seq_chunk, 1, HIDDEN), lambda i: (0, 0, 0)),
        pl.BlockSpec((1, 1, HIDDEN), lambda i: (0, 0, 0)),
        pl.BlockSpec((1, 1, HIDDEN), lambda i: (0, 0, 0)),
    ]
    kwargs = {}
    if out_alias is not None:
        args.append(out_alias)
        in_specs.append(
            pl.BlockSpec(memory_space=pltpu.MemorySpace.HBM)
        )
        kwargs["input_output_aliases"] = {4: 0}

    def body(*refs):
        _tc_add_ln_kernel(*refs[:4], refs[-1])

    return pl.pallas_call(
        body,
        grid=grid,
        in_specs=in_specs,
        out_specs=pl.BlockSpec(
            (seq_chunk, B_BLK, HIDDEN), lambda i, c=chunk_idx: (c, i, 0)
        ),
        out_shape=jax.ShapeDtypeStruct((seq, batch, HIDDEN), jnp.float32),
        compiler_params=pltpu.CompilerParams(
            dimension_semantics=("parallel",)
        ),
        **kwargs,
    )(*args)


def kernel(input_ids, word_emb, pos_emb, type_emb, gamma, beta):
    batch, seq = input_ids.shape
    n_chunks = 2
    seq_chunk = seq // n_chunks
    rows_chunk = batch * seq_chunk
    ids_sm = input_ids.astype(jnp.int32).T.reshape(1, batch * seq)
    comb = (pos_emb[:seq] + type_emb[0:1, :]).reshape(seq, 1, HIDDEN)
    gamma3 = gamma.reshape(1, 1, HIDDEN)
    beta3 = beta.reshape(1, 1, HIDDEN)

    gathered = [
        _sc_gather(
            word_emb,
            jax.lax.slice(ids_sm, (0, c * rows_chunk), (1, (c + 1) * rows_chunk)),
            rows_chunk,
        ).reshape(seq_chunk, batch, HIDDEN)
        for c in range(n_chunks)
    ]
    out = None
    for c in range(n_chunks):
        comb_c = jax.lax.slice(
            comb, (c * seq_chunk, 0, 0), ((c + 1) * seq_chunk, 1, HIDDEN)
        )
        out = _tc_add_ln_chunk(
            gathered[c], comb_c, gamma3, beta3, batch, seq, seq_chunk,
            chunk_idx=c, out_alias=out,
        )
    return jnp.transpose(out, (1, 0, 2))


# speedup vs baseline: 1.0977x; 1.0219x over previous
import jax
import jax.numpy as jnp
from jax.experimental import pallas as pl
from jax.experimental.pallas import tpu as pltpu
from jax.experimental.pallas import tpu_sc as plsc

VOCAB = 1000000
HIDDEN = 128
SEQ = 50
EPS = 1e-12

GATHER_WINDOW = 256
B_BLK = 512


def _sc_gather(word_emb, ids_flat, n_rows):
    vector_mesh = plsc.VectorSubcoreMesh(
        core_axis_name="core", subcore_axis_name="subcore"
    )

    @pl.kernel(
        out_type=jax.ShapeDtypeStruct((n_rows, HIDDEN), word_emb.dtype),
        mesh=vector_mesh,
    )
    def gather_kernel(x_hbm, i_hbm, o_hbm):
        def body(i_vmem, o_vmem):
            pltpu.sync_copy(x_hbm.at[i_vmem.at[0]], o_vmem)

        pltpu.emit_pipeline(
            body,
            grid=(n_rows // GATHER_WINDOW,),
            in_specs=[
                pl.BlockSpec((1, GATHER_WINDOW), index_map=lambda i: (0, i))
            ],
            out_specs=[
                pl.BlockSpec((GATHER_WINDOW, HIDDEN), index_map=lambda i: (i, 0))
            ],
            core_axis_name=("core", "subcore"),
            dimension_semantics=(pltpu.PARALLEL,),
        )(i_hbm, o_hbm)

    return gather_kernel(word_emb, ids_flat)


def _tc_add_ln_kernel(x_ref, comb_ref, gamma_ref, beta_ref, o_ref):
    x = x_ref[...]
    e = x + comb_ref[...]
    mean = jnp.mean(e, axis=-1, keepdims=True)
    cent = e - mean
    var = jnp.mean(cent * cent, axis=-1, keepdims=True)
    inv = jax.lax.rsqrt(var + EPS)
    o_ref[...] = cent * inv * gamma_ref[...] + beta_ref[...]


def _tc_add_ln_chunk(gathered_sm, comb, gamma, beta, batch, seq, seq_chunk,
                     chunk_idx, out_alias=None):
    grid = (batch // B_BLK,)
    args = [gathered_sm, comb, gamma, beta]
    in_specs = [
        pl.BlockSpec((seq_chunk, B_BLK, HIDDEN), lambda i: (0, i, 0)),
        pl.BlockSpec((seq_chunk, 1, HIDDEN), lambda i: (0, 0, 0)),
        pl.BlockSpec((1, 1, HIDDEN), lambda i: (0, 0, 0)),
        pl.BlockSpec((1, 1, HIDDEN), lambda i: (0, 0, 0)),
    ]
    kwargs = {}
    if out_alias is not None:
        args.append(out_alias)
        in_specs.append(
            pl.BlockSpec(memory_space=pltpu.MemorySpace.HBM)
        )
        kwargs["input_output_aliases"] = {4: 0}

    def body(*refs):
        _tc_add_ln_kernel(*refs[:4], refs[-1])

    return pl.pallas_call(
        body,
        grid=grid,
        in_specs=in_specs,
        out_specs=pl.BlockSpec(
            (seq_chunk, B_BLK, HIDDEN), lambda i, c=chunk_idx: (c, i, 0)
        ),
        out_shape=jax.ShapeDtypeStruct((seq, batch, HIDDEN), jnp.float32),
        compiler_params=pltpu.CompilerParams(
            dimension_semantics=("parallel",)
        ),
        **kwargs,
    )(*args)


def kernel(input_ids, word_emb, pos_emb, type_emb, gamma, beta):
    batch, seq = input_ids.shape
    n_chunks = 2
    seq_chunk = seq // n_chunks
    rows_chunk = batch * seq_chunk
    ids_sm = input_ids.astype(jnp.int32).T.reshape(1, batch * seq)
    comb = (pos_emb[:seq] + type_emb[0:1, :]).reshape(seq, 1, HIDDEN)
    gamma3 = gamma.reshape(1, 1, HIDDEN)
    beta3 = beta.reshape(1, 1, HIDDEN)

    gathered = [
        _sc_gather(
            word_emb,
            jax.lax.slice(ids_sm, (0, c * rows_chunk), (1, (c + 1) * rows_chunk)),
            rows_chunk,
        ).reshape(seq_chunk, batch, HIDDEN)
        for c in range(n_chunks)
    ]
    out = None
    for c in range(n_chunks):
        comb_c = jax.lax.slice(
            comb, (c * seq_chunk, 0, 0), ((c + 1) * seq_chunk, 1, HIDDEN)
        )
        out = _tc_add_ln_chunk(
            gathered[c], comb_c, gamma3, beta3, batch, seq, seq_chunk,
            chunk_idx=c, out_alias=out,
        )
    return jnp.transpose(out, (1, 0, 2))
